# baseline (speedup 1.0000x reference)
import functools

import jax
import jax.numpy as jnp
from jax import lax
from jax.experimental import pallas as pl
from jax.experimental.pallas import tpu as pltpu
from jax.experimental.pallas import tpu_sc as plsc

N_NODES = 10000
N_EDGES = 320000
D = 128

NC = 2
NS = 16
NW = NC * NS

CH = 128
NCH_TOT = 2592
NCH = NCH_TOT // NW
E_PAD = NCH_TOT * CH
NSLOT = 3


def _normalize_body(x_ref, d_ref, s_ref, ew_ref, e_ref):
    x = x_ref[...]
    norm = jnp.sqrt(jnp.sum(x * x, axis=1, keepdims=True))
    e = x / jnp.maximum(norm, 1e-12)
    e_ref[...] = e
    ew_ref[...] = e * (d_ref[...] * s_ref[0, 0])


def _make_tables(emb, d2, s2):
    return pl.pallas_call(
        _normalize_body,
        out_shape=(
            jax.ShapeDtypeStruct((N_NODES, D), jnp.float32),
            jax.ShapeDtypeStruct((N_NODES, D), jnp.float32),
        ),
    )(emb, d2, s2)


def _sc_body(ew_hbm, e_hbm, src_hbm, dst_hbm, out_hbm,
             sidx0, sidx1, sidx2, didx0, didx1, didx2,
             srows0, srows1, srows2, drows0, drows1, drows2,
             outv0, outv1, outv2,
             gsem0, gsem1, gsem2, isem0, isem1, isem2, osem0, osem1, osem2):
    wid = lax.axis_index("s") * NC + lax.axis_index("c")
    sidx = (sidx0, sidx1, sidx2)
    didx = (didx0, didx1, didx2)
    srows = (srows0, srows1, srows2)
    drows = (drows0, drows1, drows2)
    outv = (outv0, outv1, outv2)
    gsem = (gsem0, gsem1, gsem2)
    isem = (isem0, isem1, isem2)
    osem = (osem0, osem1, osem2)

    def idx_copy_async(i, s):
        h = wid + i * NW
        pltpu.async_copy(src_hbm.at[h], sidx[s], isem[s])
        pltpu.async_copy(dst_hbm.at[h], didx[s], isem[s])

    def idx_wait(s):
        pltpu.make_async_copy(src_hbm.at[0], sidx[s], isem[s]).wait()
        pltpu.make_async_copy(dst_hbm.at[0], didx[s], isem[s]).wait()

    def fire(s):
        pltpu.async_copy(ew_hbm.at[sidx[s].at[0]], srows[s], gsem[s])
        pltpu.async_copy(e_hbm.at[didx[s].at[0]], drows[s], gsem[s])

    def gather_wait(s):
        pltpu.make_async_copy(ew_hbm.at[sidx[s].at[0]], srows[s], gsem[s]).wait()
        pltpu.make_async_copy(e_hbm.at[didx[s].at[0]], drows[s], gsem[s]).wait()

    def out_write_async(i, s):
        h = wid + i * NW
        pltpu.async_copy(outv[s], out_hbm.at[pl.ds(h * CH, CH)], osem[s])

    def out_wait(s):
        pltpu.make_async_copy(outv[s], out_hbm.at[pl.ds(0, CH)], osem[s]).wait()

    for s in range(NSLOT - 1):
        idx_copy_async(s, s)
        idx_wait(s)
        fire(s)

    def outer(t, _):
        for b in range(NSLOT):
            i = t * NSLOT + b
            nxt = (b + 2) % NSLOT

            @pl.when(i + 2 < NCH)
            def _():
                idx_copy_async(i + 2, nxt)

            gather_wait(b)

            @pl.when(i >= NSLOT)
            def _():
                out_wait(b)

            def group_body(g, _):
                base = g * 16
                lane = lax.iota(jnp.int32, 16)
                res = jnp.zeros((16,), jnp.float32)
                for jj in range(16):
                    r = base + jj
                    acc = jnp.zeros((16,), jnp.float32)
                    for c2 in range(D // 16):
                        sl = pl.ds(c2 * 16, 16)
                        acc = acc + srows[b][r, sl] * drows[b][r, sl]
                    dot = jnp.sum(acc)
                    res = jnp.where(lane == jj, dot, res)
                outv[b][pl.ds(base, 16)] = res
                return 0

            lax.fori_loop(0, CH // 16, group_body, 0)
            out_write_async(i, b)

            @pl.when(i + 2 < NCH)
            def _():
                idx_wait(nxt)
                fire(nxt)
        return 0

    lax.fori_loop(0, NCH // NSLOT, outer, 0)
    for s in range(NSLOT):
        out_wait(s)


_sc_dot = functools.partial(
    pl.kernel,
    out_type=jax.ShapeDtypeStruct((E_PAD,), jnp.float32),
    mesh=plsc.VectorSubcoreMesh(
        core_axis_name="c", subcore_axis_name="s", num_cores=NC, num_subcores=NS
    ),
    scratch_types=(
        [pltpu.VMEM((1, 128), jnp.int32)] * (2 * NSLOT)
        + [pltpu.VMEM((CH, D), jnp.float32)] * (2 * NSLOT)
        + [pltpu.VMEM((CH,), jnp.float32)] * NSLOT
        + [pltpu.SemaphoreType.DMA] * (3 * NSLOT)
    ),
    compiler_params=pltpu.CompilerParams(needs_layout_passes=False),
)(_sc_body)


def kernel(emb, edge_index, d, scale):
    d2 = d.astype(jnp.float32).reshape(1, D)
    s2 = scale.astype(jnp.float32).reshape(1, 1)
    ew, e = _make_tables(emb, d2, s2)
    ei = edge_index.astype(jnp.int32)
    pad = jnp.zeros((2, E_PAD - N_EDGES), jnp.int32)
    ei = jnp.concatenate([ei, pad], axis=1)
    src = ei[0].reshape(NCH_TOT, 1, 128)
    dst = ei[1].reshape(NCH_TOT, 1, 128)
    pair = _sc_dot(ew, e, src, dst)
    return pair[:N_EDGES].reshape(N_EDGES, 1)
